# baseline (speedup 1.0000x reference)
import functools

import jax
import jax.numpy as jnp
from jax import lax
from jax.experimental import pallas as pl
from jax.experimental.pallas import tpu as pltpu
from jax.experimental.pallas import tpu_sc as plsc

MODEL_DIM = 1024
NC = 2
NS = 16
NW = NC * NS
CHUNK = 16
NBUF = 4


@functools.partial(jax.jit, static_argnames=("nchunk",))
def _sc_gather(positions, table, *, nchunk):
    b_per_w = nchunk * CHUNK
    b_total = NW * b_per_w
    ngroups = nchunk // NBUF
    brows, bcols = positions.shape
    w_per_row = bcols // b_per_w
    mesh = plsc.VectorSubcoreMesh(
        core_axis_name="c", subcore_axis_name="s", num_cores=NC, num_subcores=NS
    )

    @functools.partial(
        pl.kernel,
        out_type=jax.ShapeDtypeStruct((b_total, MODEL_DIM), jnp.float32),
        mesh=mesh,
        scratch_types=[
            pltpu.VMEM((b_per_w,), jnp.int32),
            [pltpu.VMEM((CHUNK, MODEL_DIM), jnp.float32)] * NBUF,
            [pltpu.SemaphoreType.DMA] * NBUF,
            [pltpu.SemaphoreType.DMA] * NBUF,
        ],
    )
    def k(pos_hbm, table_hbm, out_hbm, idx_v, bufs, gsems, wsems):
        wid = lax.axis_index("s") * NC + lax.axis_index("c")
        base = wid * b_per_w
        prow = wid // w_per_row
        pcol = (wid % w_per_row) * b_per_w
        pltpu.sync_copy(pos_hbm.at[prow, pl.ds(pcol, b_per_w)], idx_v)

        def issue_gather(g, b):
            pltpu.async_copy(
                table_hbm.at[idx_v.at[pl.ds(g * CHUNK, CHUNK)]], bufs[b], gsems[b]
            )

        def issue_write(g, b):
            pltpu.async_copy(
                bufs[b], out_hbm.at[pl.ds(base + g * CHUNK, CHUNK)], wsems[b]
            )

        def wait(sems, b):
            pltpu.make_async_copy(
                bufs[b], out_hbm.at[pl.ds(base, CHUNK)], sems[b]
            ).wait()

        for b in range(NBUF):
            issue_gather(b, b)

        @pl.loop(0, ngroups - 1)
        def _(go):
            for b in range(NBUF):
                wait(gsems, b)
                issue_write(go * NBUF + b, b)
            for b in range(NBUF):
                wait(wsems, b)
                issue_gather((go + 1) * NBUF + b, b)

        for b in range(NBUF):
            wait(gsems, b)
            issue_write((ngroups - 1) * NBUF + b, b)
        for b in range(NBUF):
            wait(wsems, b)

    return k(positions, table)


def kernel(positions, table):
    b, s = positions.shape
    n = b * s
    nchunk = n // (NW * CHUNK)
    out = _sc_gather(positions.astype(jnp.int32), table, nchunk=nchunk)
    return out.reshape(b, s, MODEL_DIM)


# speedup vs baseline: 1.0079x; 1.0079x over previous
import functools

import jax
import jax.numpy as jnp
from jax import lax
from jax.experimental import pallas as pl
from jax.experimental.pallas import tpu as pltpu
from jax.experimental.pallas import tpu_sc as plsc

MODEL_DIM = 1024
NC = 2
NS = 16
NW = NC * NS
CHUNK = 32
NBUF = 3
DEPTH = 1


@functools.partial(jax.jit, static_argnames=("nchunk",))
def _sc_gather(positions, table, *, nchunk):
    b_per_w = nchunk * CHUNK
    b_total = NW * b_per_w
    brows, bcols = positions.shape
    w_per_row = bcols // b_per_w
    mesh = plsc.VectorSubcoreMesh(
        core_axis_name="c", subcore_axis_name="s", num_cores=NC, num_subcores=NS
    )

    @functools.partial(
        pl.kernel,
        out_type=jax.ShapeDtypeStruct((b_total, MODEL_DIM), jnp.float32),
        mesh=mesh,
        scratch_types=[
            pltpu.VMEM((b_per_w,), jnp.int32),
            [pltpu.VMEM((CHUNK, MODEL_DIM), jnp.float32)] * NBUF,
            [pltpu.SemaphoreType.DMA] * NBUF,
            [pltpu.SemaphoreType.DMA] * NBUF,
        ],
    )
    def k(pos_hbm, table_hbm, out_hbm, idx_v, bufs, gsems, wsems):
        wid = lax.axis_index("s") * NC + lax.axis_index("c")
        base = wid * b_per_w
        prow = wid // w_per_row
        pcol = (wid % w_per_row) * b_per_w
        pltpu.sync_copy(pos_hbm.at[prow, pl.ds(pcol, b_per_w)], idx_v)
        gathers = [None] * NBUF
        pending_w = [None] * NBUF

        def issue_gather(p):
            pb = p % NBUF
            if pending_w[pb] is not None:
                pending_w[pb].wait()
                pending_w[pb] = None
            gathers[pb] = pltpu.async_copy(
                table_hbm.at[idx_v.at[pl.ds(p * CHUNK, CHUNK)]], bufs[pb], gsems[pb]
            )

        for p in range(min(DEPTH, nchunk)):
            issue_gather(p)
        for g in range(nchunk):
            b = g % NBUF
            p = g + DEPTH
            if p < nchunk:
                issue_gather(p)
            gathers[b].wait()
            pending_w[b] = pltpu.async_copy(
                bufs[b], out_hbm.at[pl.ds(base + g * CHUNK, CHUNK)], wsems[b]
            )
        for b in range(NBUF):
            if pending_w[b] is not None:
                pending_w[b].wait()

    return k(positions, table)


def kernel(positions, table):
    b, s = positions.shape
    n = b * s
    nchunk = n // (NW * CHUNK)
    out = _sc_gather(positions.astype(jnp.int32), table, nchunk=nchunk)
    return out.reshape(b, s, MODEL_DIM)


# speedup vs baseline: 1.0154x; 1.0074x over previous
import functools

import jax
import jax.numpy as jnp
from jax import lax
from jax.experimental import pallas as pl
from jax.experimental.pallas import tpu as pltpu
from jax.experimental.pallas import tpu_sc as plsc

MODEL_DIM = 1024
NC = 2
NS = 16
NW = NC * NS
CHUNK = 16
NBUF = 7
DEPTH = 3


@functools.partial(jax.jit, static_argnames=("nchunk",))
def _sc_gather(positions, table, *, nchunk):
    b_per_w = nchunk * CHUNK
    b_total = NW * b_per_w
    brows, bcols = positions.shape
    w_per_row = bcols // b_per_w
    mesh = plsc.VectorSubcoreMesh(
        core_axis_name="c", subcore_axis_name="s", num_cores=NC, num_subcores=NS
    )

    @functools.partial(
        pl.kernel,
        out_type=jax.ShapeDtypeStruct((b_total, MODEL_DIM), jnp.float32),
        mesh=mesh,
        scratch_types=[
            pltpu.VMEM((b_per_w,), jnp.int32),
            [pltpu.VMEM((CHUNK, MODEL_DIM), jnp.float32)] * NBUF,
            [pltpu.SemaphoreType.DMA] * NBUF,
            [pltpu.SemaphoreType.DMA] * NBUF,
        ],
    )
    def k(pos_hbm, table_hbm, out_hbm, idx_v, bufs, gsems, wsems):
        wid = lax.axis_index("s") * NC + lax.axis_index("c")
        base = wid * b_per_w
        prow = wid // w_per_row
        pcol = (wid % w_per_row) * b_per_w
        pltpu.sync_copy(pos_hbm.at[prow, pl.ds(pcol, 128)], idx_v.at[pl.ds(0, 128)])
        gathers = [None] * NBUF
        pending_w = [None] * NBUF

        def issue_gather(p):
            pb = p % NBUF
            if pending_w[pb] is not None:
                pending_w[pb].wait()
                pending_w[pb] = None
            gathers[pb] = pltpu.async_copy(
                table_hbm.at[idx_v.at[pl.ds(p * CHUNK, CHUNK)]], bufs[pb], gsems[pb]
            )

        for p in range(min(DEPTH, nchunk)):
            issue_gather(p)
        pltpu.sync_copy(
            pos_hbm.at[prow, pl.ds(pcol + 128, b_per_w - 128)],
            idx_v.at[pl.ds(128, b_per_w - 128)],
        )
        for g in range(nchunk):
            b = g % NBUF
            p = g + DEPTH
            if p < nchunk:
                issue_gather(p)
            gathers[b].wait()
            pending_w[b] = pltpu.async_copy(
                bufs[b], out_hbm.at[pl.ds(base + g * CHUNK, CHUNK)], wsems[b]
            )
        for b in range(NBUF):
            if pending_w[b] is not None:
                pending_w[b].wait()

    return k(positions, table)


def kernel(positions, table):
    b, s = positions.shape
    n = b * s
    nchunk = n // (NW * CHUNK)
    out = _sc_gather(positions.astype(jnp.int32), table, nchunk=nchunk)
    return out.reshape(b, s, MODEL_DIM)


# speedup vs baseline: 1.0200x; 1.0045x over previous
import functools

import jax
import jax.numpy as jnp
from jax import lax
from jax.experimental import pallas as pl
from jax.experimental.pallas import tpu as pltpu
from jax.experimental.pallas import tpu_sc as plsc

MODEL_DIM = 1024
NC = 2
NS = 16
NW = NC * NS
CHUNK = 16
NBUF = 7
DEPTH = 3


@functools.partial(jax.jit, static_argnames=("nchunk",))
def _sc_gather(positions, table, *, nchunk):
    b_per_w = nchunk * CHUNK
    b_total = NW * b_per_w
    brows, bcols = positions.shape
    w_per_row = bcols // b_per_w
    mesh = plsc.VectorSubcoreMesh(
        core_axis_name="c", subcore_axis_name="s", num_cores=NC, num_subcores=NS
    )

    @functools.partial(
        pl.kernel,
        out_type=jax.ShapeDtypeStruct((b_total, MODEL_DIM), jnp.float32),
        mesh=mesh,
        scratch_types=[
            pltpu.VMEM((b_per_w,), jnp.int32),
            [pltpu.VMEM((CHUNK, MODEL_DIM), jnp.float32)] * NBUF,
            [pltpu.SemaphoreType.DMA] * NBUF,
            [pltpu.SemaphoreType.DMA] * NBUF,
        ],
    )
    def k(pos_hbm, table_hbm, out_hbm, idx_v, bufs, gsems, wsems):
        wid = lax.axis_index("s") * NC + lax.axis_index("c")
        base = wid * b_per_w
        prow = wid // w_per_row
        pcol = (wid % w_per_row) * b_per_w
        pltpu.sync_copy(pos_hbm.at[prow, pl.ds(pcol, b_per_w)], idx_v)
        gathers = [None] * NBUF
        pending_w = [None] * NBUF

        def issue_gather(p):
            pb = p % NBUF
            if pending_w[pb] is not None:
                pending_w[pb].wait()
                pending_w[pb] = None
            gathers[pb] = pltpu.async_copy(
                table_hbm.at[idx_v.at[pl.ds(p * CHUNK, CHUNK)]], bufs[pb], gsems[pb]
            )

        for p in range(min(DEPTH, nchunk)):
            issue_gather(p)
        for g in range(nchunk):
            b = g % NBUF
            p = g + DEPTH
            if p < nchunk:
                issue_gather(p)
            gathers[b].wait()
            pending_w[b] = pltpu.async_copy(
                bufs[b], out_hbm.at[pl.ds(base + g * CHUNK, CHUNK)], wsems[b]
            )
        for b in range(NBUF):
            if pending_w[b] is not None:
                pending_w[b].wait()

    return k(positions, table)


def kernel(positions, table):
    b, s = positions.shape
    n = b * s
    nchunk = n // (NW * CHUNK)
    out = _sc_gather(positions.astype(jnp.int32), table, nchunk=nchunk)
    return out.reshape(b, s, MODEL_DIM)
